# baseline (speedup 1.0000x reference)
import jax, jax.numpy as jnp
from jax.experimental import pallas as pl

def kernel(node_embedding_matrix, batch_x_index, W1, b1, W2, b2):
    ge = jax.ops.segment_sum(node_embedding_matrix, batch_x_index, num_segments=4096)
    h = jax.nn.relu(jnp.einsum('gd,tde->tge', ge, W1) + b1[:, None, :])
    pred = jnp.einsum('tge,teo->tgo', h, W2) + b2[:, None, :]
    return jnp.transpose(pred, (1, 2, 0))


# speedup vs baseline: 3.5159x; 3.5159x over previous
import jax
import jax.numpy as jnp
from jax import lax
from jax.experimental import pallas as pl
from jax.experimental.pallas import tpu as pltpu
from jax.experimental.pallas import tpu_sc as plsc

N = 50000
D = 512
G = 4096
T = 8
C = 64

NC = 2
NS = 16
NP = 2 * NS
DH = D // NC
NV = DH // 16
CH = 80
NCHUNK = N // CH
GB = G // NS

_sc_mesh = plsc.VectorSubcoreMesh(core_axis_name="c", subcore_axis_name="s")


_sc_scratch = [
    pltpu.VMEM((CH, DH), jnp.float32),
    pltpu.VMEM((CH,), jnp.int32),
    pltpu.VMEM((DH,), jnp.float32),
    pltpu.VMEM((16,), jnp.int32),
    pltpu.VMEM_SHARED((G, DH), jnp.float32),
]

_sc_out_types = (
    jax.ShapeDtypeStruct((G, D), jnp.float32),
    jax.ShapeDtypeStruct((NP, D), jnp.float32),
    jax.ShapeDtypeStruct((NP, 16), jnp.int32),
)


def _segment_sum_body(
    nodes_hbm, idx_hbm, zeros_hbm,
    out_hbm, parts_hbm, pids_hbm,
    buf_v, idx_v, stage_v, idwr_v,
    acc_sh,
):
    c = lax.axis_index("c")
    s = lax.axis_index("s")
    col0 = c * DH

    idwr_v[...] = jnp.full((16,), -1, jnp.int32)
    pltpu.sync_copy(idwr_v, pids_hbm.at[2 * s])
    pltpu.sync_copy(zeros_hbm, acc_sh.at[pl.ds(s * GB, GB)])
    plsc.subcore_barrier()

    nchunks = jnp.where(s == 0, NCHUNK - (NS - 1) * (NCHUNK // NS), NCHUNK // NS)
    chunk0 = jnp.where(s == 0, 0, (NCHUNK // NS) * s + 1)

    def flush(cur_g, acc, first_done):
        for k in range(NV):
            stage_v[pl.ds(16 * k, 16)] = acc[k]

        @pl.when(first_done == 0)
        def _():
            idwr_v[...] = jnp.full((16,), cur_g, jnp.int32)
            pltpu.sync_copy(stage_v, parts_hbm.at[2 * s, pl.ds(col0, DH)])
            pltpu.sync_copy(idwr_v, pids_hbm.at[2 * s])

        @pl.when(first_done != 0)
        def _():
            pltpu.sync_copy(stage_v, acc_sh.at[cur_g])

    def group_body(q, carry):
        gvec = idx_v[pl.ds(q * 16, 16)]
        for i in range(16):
            cur_g, first_done, acc = carry
            r = q * 16 + i
            g = gvec[i]
            row = tuple(buf_v[r, pl.ds(16 * k, 16)] for k in range(NV))
            same = g == cur_g

            @pl.when(jnp.logical_and(jnp.logical_not(same), cur_g >= 0))
            def _():
                flush(cur_g, acc, first_done)

            new_acc = tuple(
                jnp.where(same, acc[k] + row[k], row[k]) for k in range(NV)
            )
            new_first = jnp.where(
                jnp.logical_or(same, cur_g < 0), first_done, 1
            )
            carry = (g, new_first, new_acc)
        return carry

    def chunk_body(j, carry):
        base = (chunk0 + j) * CH
        pltpu.sync_copy(idx_hbm.at[pl.ds(base, CH)], idx_v)
        pltpu.sync_copy(nodes_hbm.at[pl.ds(base, CH), pl.ds(col0, DH)], buf_v)
        return lax.fori_loop(0, CH // 16, group_body, carry)

    zero_acc = tuple(jnp.zeros((16,), jnp.float32) for _ in range(NV))
    cur_g, first_done, acc = lax.fori_loop(
        0, nchunks, chunk_body, (jnp.int32(-1), jnp.int32(0), zero_acc)
    )

    for k in range(NV):
        stage_v[pl.ds(16 * k, 16)] = acc[k]
    idwr_v[...] = jnp.full((16,), cur_g, jnp.int32)
    pltpu.sync_copy(stage_v, parts_hbm.at[2 * s + 1, pl.ds(col0, DH)])
    pltpu.sync_copy(idwr_v, pids_hbm.at[2 * s + 1])

    plsc.subcore_barrier()
    pltpu.sync_copy(
        acc_sh.at[pl.ds(s * GB, GB)],
        out_hbm.at[pl.ds(s * GB, GB), pl.ds(col0, DH)],
    )


def _make_segment_sum(interpret=False):
    return pl.kernel(
        _segment_sum_body,
        out_type=_sc_out_types,
        mesh=_sc_mesh,
        scratch_types=_sc_scratch,
        interpret=interpret,
    )


_segment_sum_sc = _make_segment_sum()


BG = 1024


def _mlp_body(ge_ref, oh_ref, parts_ref, w1_ref, b1_ref, w2_ref, b2_ref,
              out_ref):
    corr = lax.dot_general(
        oh_ref[...], parts_ref[...],
        (((0,), (0,)), ((), ())),
        preferred_element_type=jnp.float32,
    )
    ge = ge_ref[...] + corr
    h = jnp.dot(ge, w1_ref[0], preferred_element_type=jnp.float32)
    h = jnp.maximum(h + b1_ref[0], 0.0)
    p = jnp.dot(h, w2_ref[0], preferred_element_type=jnp.float32) + b2_ref[0]
    out_ref[0] = p


def _mlp_tc(ge, onehot, parts, W1, b1, W2, b2):
    grid = (G // BG, T)
    return pl.pallas_call(
        _mlp_body,
        grid=grid,
        in_specs=[
            pl.BlockSpec((BG, D), lambda g, t: (g, 0)),
            pl.BlockSpec((NP, BG), lambda g, t: (0, g)),
            pl.BlockSpec((NP, D), lambda g, t: (0, 0)),
            pl.BlockSpec((1, D, D), lambda g, t: (t, 0, 0)),
            pl.BlockSpec((1, 1, D), lambda g, t: (t, 0, 0)),
            pl.BlockSpec((1, D, C), lambda g, t: (t, 0, 0)),
            pl.BlockSpec((1, 1, C), lambda g, t: (t, 0, 0)),
        ],
        out_specs=pl.BlockSpec((1, BG, C), lambda g, t: (t, g, 0)),
        out_shape=jax.ShapeDtypeStruct((T, G, C), jnp.float32),
    )(ge, onehot, parts, W1, b1.reshape(T, 1, D), W2, b2.reshape(T, 1, C))


def kernel(node_embedding_matrix, batch_x_index, W1, b1, W2, b2):
    idx = batch_x_index.astype(jnp.int32)
    zeros = jnp.zeros((GB, DH), jnp.float32)
    ge, parts, pids = _segment_sum_sc(node_embedding_matrix, idx, zeros)
    ids = pids[:, 0]
    onehot = (ids[:, None] == jnp.arange(G, dtype=jnp.int32)[None, :]).astype(
        jnp.float32
    )
    pred = _mlp_tc(ge, onehot, parts, W1, b1, W2, b2)
    return jnp.transpose(pred, (1, 2, 0))


# speedup vs baseline: 4.2653x; 1.2131x over previous
import jax
import jax.numpy as jnp
from jax import lax
from jax.experimental import pallas as pl
from jax.experimental.pallas import tpu as pltpu
from jax.experimental.pallas import tpu_sc as plsc

N = 50000
D = 512
G = 4096
T = 8
C = 64

NC = 2
NS = 16
NP = 2 * NS
DH = D // NC
NV = DH // 16
CH = 80
NCHUNK = N // CH
GB = G // NS

_sc_mesh = plsc.VectorSubcoreMesh(core_axis_name="c", subcore_axis_name="s")


_sc_scratch = [
    pltpu.VMEM((CH, DH), jnp.float32),
    pltpu.VMEM((CH,), jnp.int32),
    pltpu.VMEM((CH, DH), jnp.float32),
    pltpu.VMEM((CH,), jnp.int32),
    pltpu.VMEM((DH,), jnp.float32),
    pltpu.VMEM((16,), jnp.int32),
    pltpu.VMEM_SHARED((G, DH), jnp.float32),
    pltpu.SemaphoreType.DMA,
    pltpu.SemaphoreType.DMA,
]

_sc_out_types = (
    jax.ShapeDtypeStruct((G, D), jnp.float32),
    jax.ShapeDtypeStruct((NP, D), jnp.float32),
    jax.ShapeDtypeStruct((NP, 16), jnp.int32),
)


def _segment_sum_body(
    nodes_hbm, idx_hbm, zeros_hbm,
    out_hbm, parts_hbm, pids_hbm,
    buf_a, idx_a, buf_b, idx_b, stage_v, idwr_v,
    acc_sh, sem_a, sem_b,
):
    c = lax.axis_index("c")
    s = lax.axis_index("s")
    col0 = c * DH

    idwr_v[...] = jnp.full((16,), -1, jnp.int32)
    pltpu.sync_copy(idwr_v, pids_hbm.at[2 * s])
    pltpu.sync_copy(zeros_hbm, acc_sh.at[pl.ds(s * GB, GB)])
    plsc.subcore_barrier()

    per = NCHUNK // NS
    chunk0 = per * s
    nchunks = jnp.where(s == NS - 1, per + 1, per)

    def start_chunk(j, buf, idxb, sem):
        base = (chunk0 + jnp.minimum(j, per)) * CH
        pltpu.async_copy(idx_hbm.at[pl.ds(base, CH)], idxb, sem)
        pltpu.async_copy(
            nodes_hbm.at[pl.ds(base, CH), pl.ds(col0, DH)], buf, sem
        )

    def wait_chunk(buf, idxb, sem):
        pltpu.make_async_copy(idx_hbm.at[pl.ds(0, CH)], idxb, sem).wait()
        pltpu.make_async_copy(
            nodes_hbm.at[pl.ds(0, CH), pl.ds(0, DH)], buf, sem
        ).wait()

    def flush(cur_g, acc, first_done):
        for k in range(NV):
            stage_v[pl.ds(16 * k, 16)] = acc[k]

        @pl.when(first_done == 0)
        def _():
            idwr_v[...] = jnp.full((16,), cur_g, jnp.int32)
            pltpu.sync_copy(stage_v, parts_hbm.at[2 * s, pl.ds(col0, DH)])
            pltpu.sync_copy(idwr_v, pids_hbm.at[2 * s])

        @pl.when(first_done != 0)
        def _():
            pltpu.sync_copy(stage_v, acc_sh.at[cur_g])

    def process(buf, idxb, live, carry):
        def group_body(q, carry):
            gvec = idxb[pl.ds(q * 16, 16)]
            for i in range(16):
                cur_g, first_done, acc = carry
                r = q * 16 + i
                g = gvec[i]
                row = tuple(buf[r, pl.ds(16 * k, 16)] for k in range(NV))
                same = g == cur_g

                @pl.when(
                    jnp.logical_and(
                        live,
                        jnp.logical_and(jnp.logical_not(same), cur_g >= 0),
                    )
                )
                def _():
                    flush(cur_g, acc, first_done)

                new_acc = tuple(
                    jnp.where(same, acc[k] + row[k], row[k])
                    for k in range(NV)
                )
                new_first = jnp.where(
                    jnp.logical_or(same, cur_g < 0), first_done, 1
                )
                carry = (
                    jnp.where(live, g, cur_g),
                    jnp.where(live, new_first, first_done),
                    tuple(
                        jnp.where(live, new_acc[k], acc[k]) for k in range(NV)
                    ),
                )
            return carry

        return lax.fori_loop(0, CH // 16, group_body, carry)

    def pair_body(p, carry):
        ja = 2 * p
        jb = ja + 1
        start_chunk(jb, buf_b, idx_b, sem_b)
        wait_chunk(buf_a, idx_a, sem_a)
        carry = process(buf_a, idx_a, ja < nchunks, carry)
        start_chunk(ja + 2, buf_a, idx_a, sem_a)
        wait_chunk(buf_b, idx_b, sem_b)
        return process(buf_b, idx_b, jb < nchunks, carry)

    zero_acc = tuple(jnp.zeros((16,), jnp.float32) for _ in range(NV))
    start_chunk(0, buf_a, idx_a, sem_a)
    cur_g, first_done, acc = lax.fori_loop(
        0, (per + 1) // 2, pair_body, (jnp.int32(-1), jnp.int32(0), zero_acc)
    )
    wait_chunk(buf_a, idx_a, sem_a)

    for k in range(NV):
        stage_v[pl.ds(16 * k, 16)] = acc[k]
    idwr_v[...] = jnp.full((16,), cur_g, jnp.int32)
    pltpu.sync_copy(stage_v, parts_hbm.at[2 * s + 1, pl.ds(col0, DH)])
    pltpu.sync_copy(idwr_v, pids_hbm.at[2 * s + 1])

    plsc.subcore_barrier()
    pltpu.sync_copy(
        acc_sh.at[pl.ds(s * GB, GB)],
        out_hbm.at[pl.ds(s * GB, GB), pl.ds(col0, DH)],
    )


def _make_segment_sum(interpret=False):
    return pl.kernel(
        _segment_sum_body,
        out_type=_sc_out_types,
        mesh=_sc_mesh,
        scratch_types=_sc_scratch,
        interpret=interpret,
    )


_segment_sum_sc = _make_segment_sum()


BG = 1024


def _mlp_body(ge_ref, oh_ref, parts_ref, w1_ref, b1_ref, w2_ref, b2_ref,
              out_ref):
    corr = lax.dot_general(
        oh_ref[...], parts_ref[...],
        (((0,), (0,)), ((), ())),
        preferred_element_type=jnp.float32,
    )
    ge = ge_ref[...] + corr
    h = jnp.dot(ge, w1_ref[0], preferred_element_type=jnp.float32)
    h = jnp.maximum(h + b1_ref[0], 0.0)
    p = jnp.dot(h, w2_ref[0], preferred_element_type=jnp.float32) + b2_ref[0]
    out_ref[0] = p


def _mlp_tc(ge, onehot, parts, W1, b1, W2, b2):
    grid = (G // BG, T)
    return pl.pallas_call(
        _mlp_body,
        grid=grid,
        in_specs=[
            pl.BlockSpec((BG, D), lambda g, t: (g, 0)),
            pl.BlockSpec((NP, BG), lambda g, t: (0, g)),
            pl.BlockSpec((NP, D), lambda g, t: (0, 0)),
            pl.BlockSpec((1, D, D), lambda g, t: (t, 0, 0)),
            pl.BlockSpec((1, 1, D), lambda g, t: (t, 0, 0)),
            pl.BlockSpec((1, D, C), lambda g, t: (t, 0, 0)),
            pl.BlockSpec((1, 1, C), lambda g, t: (t, 0, 0)),
        ],
        out_specs=pl.BlockSpec((1, BG, C), lambda g, t: (t, g, 0)),
        out_shape=jax.ShapeDtypeStruct((T, G, C), jnp.float32),
    )(ge, onehot, parts, W1, b1.reshape(T, 1, D), W2, b2.reshape(T, 1, C))


def kernel(node_embedding_matrix, batch_x_index, W1, b1, W2, b2):
    idx = batch_x_index.astype(jnp.int32)
    zeros = jnp.zeros((GB, DH), jnp.float32)
    ge, parts, pids = _segment_sum_sc(node_embedding_matrix, idx, zeros)
    ids = pids[:, 0]
    onehot = (ids[:, None] == jnp.arange(G, dtype=jnp.int32)[None, :]).astype(
        jnp.float32
    )
    pred = _mlp_tc(ge, onehot, parts, W1, b1, W2, b2)
    return jnp.transpose(pred, (1, 2, 0))


# speedup vs baseline: 4.2670x; 1.0004x over previous
import jax
import jax.numpy as jnp
from jax import lax
from jax.experimental import pallas as pl
from jax.experimental.pallas import tpu as pltpu
from jax.experimental.pallas import tpu_sc as plsc

N = 50000
D = 512
G = 4096
T = 8
C = 64

NC = 2
NS = 16
NP = 2 * NS
DH = D // NC
NV = DH // 16
CH = 80
NCHUNK = N // CH
GB = G // NS

_sc_mesh = plsc.VectorSubcoreMesh(core_axis_name="c", subcore_axis_name="s")


_sc_scratch = [
    pltpu.VMEM((CH, DH), jnp.float32),
    pltpu.VMEM((CH,), jnp.int32),
    pltpu.VMEM((CH, DH), jnp.float32),
    pltpu.VMEM((CH,), jnp.int32),
    pltpu.VMEM((DH,), jnp.float32),
    pltpu.VMEM((16,), jnp.int32),
    pltpu.VMEM_SHARED((G, DH), jnp.float32),
    pltpu.SemaphoreType.DMA,
    pltpu.SemaphoreType.DMA,
]

_sc_out_types = (
    jax.ShapeDtypeStruct((G, D), jnp.float32),
    jax.ShapeDtypeStruct((NP, D), jnp.float32),
    jax.ShapeDtypeStruct((NP, 16), jnp.int32),
)


def _segment_sum_body(
    nodes_hbm, idx_hbm, zeros_hbm,
    out_hbm, parts_hbm, pids_hbm,
    buf_a, idx_a, buf_b, idx_b, stage_v, idwr_v,
    acc_sh, sem_a, sem_b,
):
    c = lax.axis_index("c")
    s = lax.axis_index("s")
    col0 = c * DH

    idwr_v[...] = jnp.full((16,), -1, jnp.int32)
    pltpu.sync_copy(idwr_v, pids_hbm.at[2 * s])
    pltpu.sync_copy(zeros_hbm, acc_sh.at[pl.ds(s * GB, GB)])
    plsc.subcore_barrier()

    per = NCHUNK // NS
    chunk0 = per * s
    nchunks = jnp.where(s == NS - 1, per + 1, per)

    def start_chunk(j, buf, idxb, sem):
        base = (chunk0 + jnp.minimum(j, per)) * CH
        pltpu.async_copy(idx_hbm.at[pl.ds(base, CH)], idxb, sem)
        pltpu.async_copy(
            nodes_hbm.at[pl.ds(base, CH), pl.ds(col0, DH)], buf, sem
        )

    def wait_chunk(buf, idxb, sem):
        pltpu.make_async_copy(idx_hbm.at[pl.ds(0, CH)], idxb, sem).wait()
        pltpu.make_async_copy(
            nodes_hbm.at[pl.ds(0, CH), pl.ds(0, DH)], buf, sem
        ).wait()

    def flush(cur_g, acc, first_done):
        for k in range(NV):
            stage_v[pl.ds(16 * k, 16)] = acc[k]

        @pl.when(first_done == 0)
        def _():
            idwr_v[...] = jnp.full((16,), cur_g, jnp.int32)
            pltpu.sync_copy(stage_v, parts_hbm.at[2 * s, pl.ds(col0, DH)])
            pltpu.sync_copy(idwr_v, pids_hbm.at[2 * s])

        @pl.when(first_done != 0)
        def _():
            pltpu.sync_copy(stage_v, acc_sh.at[cur_g])

    def process(buf, idxb, live, carry):
        def group_body(q, carry):
            gvec = idxb[pl.ds(q * 16, 16)]
            for i in range(16):
                cur_g, first_done, acc = carry
                r = q * 16 + i
                g = gvec[i]
                row = tuple(buf[r, pl.ds(16 * k, 16)] for k in range(NV))
                same = g == cur_g

                @pl.when(
                    jnp.logical_and(
                        live,
                        jnp.logical_and(jnp.logical_not(same), cur_g >= 0),
                    )
                )
                def _():
                    flush(cur_g, acc, first_done)

                new_acc = tuple(
                    jnp.where(same, acc[k] + row[k], row[k])
                    for k in range(NV)
                )
                new_first = jnp.where(
                    jnp.logical_or(same, cur_g < 0), first_done, 1
                )
                carry = (
                    jnp.where(live, g, cur_g),
                    jnp.where(live, new_first, first_done),
                    tuple(
                        jnp.where(live, new_acc[k], acc[k]) for k in range(NV)
                    ),
                )
            return carry

        return lax.fori_loop(0, CH // 16, group_body, carry)

    def pair_body(p, carry):
        ja = 2 * p
        jb = ja + 1
        start_chunk(jb, buf_b, idx_b, sem_b)
        wait_chunk(buf_a, idx_a, sem_a)
        carry = process(buf_a, idx_a, ja < nchunks, carry)
        start_chunk(ja + 2, buf_a, idx_a, sem_a)
        wait_chunk(buf_b, idx_b, sem_b)
        return process(buf_b, idx_b, jb < nchunks, carry)

    zero_acc = tuple(jnp.zeros((16,), jnp.float32) for _ in range(NV))
    start_chunk(0, buf_a, idx_a, sem_a)
    cur_g, first_done, acc = lax.fori_loop(
        0, (per + 1) // 2, pair_body, (jnp.int32(-1), jnp.int32(0), zero_acc)
    )
    wait_chunk(buf_a, idx_a, sem_a)

    for k in range(NV):
        stage_v[pl.ds(16 * k, 16)] = acc[k]
    idwr_v[...] = jnp.full((16,), cur_g, jnp.int32)
    pltpu.sync_copy(stage_v, parts_hbm.at[2 * s + 1, pl.ds(col0, DH)])
    pltpu.sync_copy(idwr_v, pids_hbm.at[2 * s + 1])

    plsc.subcore_barrier()
    pltpu.sync_copy(
        acc_sh.at[pl.ds(s * GB, GB)],
        out_hbm.at[pl.ds(s * GB, GB), pl.ds(col0, DH)],
    )


def _make_segment_sum(interpret=False):
    return pl.kernel(
        _segment_sum_body,
        out_type=_sc_out_types,
        mesh=_sc_mesh,
        scratch_types=_sc_scratch,
        interpret=interpret,
    )


_segment_sum_sc = _make_segment_sum()


BG = 1024


def _mlp_body(ge_ref, oh_ref, parts_ref, w1_ref, b1_ref, w2_ref, b2_ref,
              out_ref):
    corr = lax.dot_general(
        oh_ref[...], parts_ref[...],
        (((0,), (0,)), ((), ())),
        preferred_element_type=jnp.float32,
    )
    ge = (ge_ref[...] + corr).astype(jnp.bfloat16)
    h = jnp.dot(ge, w1_ref[0], preferred_element_type=jnp.float32)
    h = jnp.maximum(h + b1_ref[0], 0.0).astype(jnp.bfloat16)
    p = jnp.dot(h, w2_ref[0], preferred_element_type=jnp.float32) + b2_ref[0]
    out_ref[0] = p


def _mlp_tc(ge, onehot, parts, W1, b1, W2, b2):
    grid = (G // BG, T)
    return pl.pallas_call(
        _mlp_body,
        grid=grid,
        in_specs=[
            pl.BlockSpec((BG, D), lambda g, t: (g, 0)),
            pl.BlockSpec((NP, BG), lambda g, t: (0, g)),
            pl.BlockSpec((NP, D), lambda g, t: (0, 0)),
            pl.BlockSpec((1, D, D), lambda g, t: (t, 0, 0)),
            pl.BlockSpec((1, 1, D), lambda g, t: (t, 0, 0)),
            pl.BlockSpec((1, D, C), lambda g, t: (t, 0, 0)),
            pl.BlockSpec((1, 1, C), lambda g, t: (t, 0, 0)),
        ],
        out_specs=pl.BlockSpec((1, BG, C), lambda g, t: (t, g, 0)),
        out_shape=jax.ShapeDtypeStruct((T, G, C), jnp.float32),
    )(
        ge, onehot, parts,
        W1.astype(jnp.bfloat16), b1.reshape(T, 1, D),
        W2.astype(jnp.bfloat16), b2.reshape(T, 1, C),
    )


def kernel(node_embedding_matrix, batch_x_index, W1, b1, W2, b2):
    idx = batch_x_index.astype(jnp.int32)
    zeros = jnp.zeros((GB, DH), jnp.float32)
    ge, parts, pids = _segment_sum_sc(node_embedding_matrix, idx, zeros)
    ids = pids[:, 0]
    onehot = (ids[:, None] == jnp.arange(G, dtype=jnp.int32)[None, :]).astype(
        jnp.float32
    )
    pred = _mlp_tc(ge, onehot, parts, W1, b1, W2, b2)
    return jnp.transpose(pred, (1, 2, 0))
